# baseline (speedup 1.0000x reference)
import functools

import jax
import jax.numpy as jnp
from jax import lax
from jax.experimental import pallas as pl
from jax.experimental.pallas import tpu as pltpu
from jax.experimental.pallas import tpu_sc as plsc

N = 10000
E = 320000
D = 128
H = 256

NC = 2
NS = 16
NW = NC * NS

P = 2
EC = E // P
EPT = EC // NW
CHUNK = 40
NCHUNK = EPT // CHUNK
EPTG = E // NW
NCHUNKG = EPTG // CHUNK
SBLK = 40
SCPB = SBLK // CHUNK
NSBLK = EPT // SBLK
SLAB = 624
TAIL = N - SLAB * NS


def _pack_pair(y):
    yb = y.astype(jnp.bfloat16)
    lo = jax.lax.bitcast_convert_type(yb[:, :D], jnp.uint16).astype(jnp.uint32)
    hi = jax.lax.bitcast_convert_type(yb[:, D:], jnp.uint16).astype(jnp.uint32)
    return jax.lax.bitcast_convert_type(lo | (hi << 16), jnp.int32)


def _pre_body(x_ref, ws_ref, wr_ref, xs_ref, xr_ref):
    xb = x_ref[...]
    xs_ref[...] = _pack_pair(
        jnp.dot(xb, ws_ref[...], preferred_element_type=jnp.float32))
    xr_ref[...] = _pack_pair(
        jnp.dot(xb, wr_ref[...], preferred_element_type=jnp.float32))


def _pre_project(x, ws, wr):
    blk = 1000
    return pl.pallas_call(
        _pre_body,
        grid=(N // blk,),
        in_specs=[
            pl.BlockSpec((blk, D), lambda i: (i, 0)),
            pl.BlockSpec((D, H), lambda i: (0, 0)),
            pl.BlockSpec((D, H), lambda i: (0, 0)),
        ],
        out_specs=[
            pl.BlockSpec((blk, D), lambda i: (i, 0)),
            pl.BlockSpec((blk, D), lambda i: (i, 0)),
        ],
        out_shape=[
            jax.ShapeDtypeStruct((N, D), jnp.int32),
            jax.ShapeDtypeStruct((N, D), jnp.int32),
        ],
    )(x, ws, wr)


NSLOT = 4


def _gather_body(xs_hbm, xr_hbm, sidx_hbm, ridx_hbm, outs_hbm, outr_hbm,
                 sidx_all, ridx_all,
                 ba0, bb0, ba1, bb1, ba2, bb2, ba3, bb3,
                 sa0, sb0, soa0, sob0, sa1, sb1, soa1, sob1,
                 sa2, sb2, soa2, sob2, sa3, sb3, soa3, sob3):
    c = lax.axis_index("c")
    s = lax.axis_index("s")
    wid = s * NC + c
    tile_base = wid * EPTG
    pltpu.sync_copy(sidx_hbm.at[wid], sidx_all)
    pltpu.sync_copy(ridx_hbm.at[wid], ridx_all)

    slots = ((ba0, bb0, sa0, sb0, soa0, sob0),
             (ba1, bb1, sa1, sb1, soa1, sob1),
             (ba2, bb2, sa2, sb2, soa2, sob2),
             (ba3, bb3, sa3, sb3, soa3, sob3))

    def issue_gather(q, sl):
        ba, bb, sa, sb, _, _ = slots[sl]
        pltpu.async_copy(xs_hbm.at[sidx_all.at[q]], ba, sa)
        pltpu.async_copy(xr_hbm.at[ridx_all.at[q]], bb, sb)

    def wait_gather(sl):
        ba, bb, sa, sb, _, _ = slots[sl]
        pltpu.make_async_copy(xs_hbm.at[sidx_all.at[0]], ba, sa).wait()
        pltpu.make_async_copy(xr_hbm.at[ridx_all.at[0]], bb, sb).wait()

    def issue_out(q, sl):
        ba, bb, _, _, soa, sob = slots[sl]
        base = pl.multiple_of(tile_base + q * CHUNK, CHUNK)
        pltpu.async_copy(ba, outs_hbm.at[pl.ds(base, CHUNK)], soa)
        pltpu.async_copy(bb, outr_hbm.at[pl.ds(base, CHUNK)], sob)

    def wait_out(sl):
        ba, bb, _, _, soa, sob = slots[sl]
        pltpu.make_async_copy(ba, outs_hbm.at[pl.ds(0, CHUNK)], soa).wait()
        pltpu.make_async_copy(bb, outr_hbm.at[pl.ds(0, CHUNK)], sob).wait()

    for q0 in range(NSLOT - 1):
        issue_gather(q0, q0)

    def group(t, carry):
        for s4 in range(NSLOT):
            q = NSLOT * t + s4
            prev = (s4 - 1) % NSLOT

            @pl.when(q < NCHUNKG)
            def _visit():
                wait_gather(s4)
                issue_out(q, s4)

                @pl.when((q >= 1) & (q + NSLOT - 1 < NCHUNKG))
                def _reissue():
                    wait_out(prev)
                    issue_gather(q + NSLOT - 1, prev)

                @pl.when(q == 0)
                def _first():
                    issue_gather(NSLOT - 1, prev)

        return carry

    lax.fori_loop(0, (NCHUNKG + NSLOT - 1) // NSLOT, group, 0, unroll=False)
    for sl in range(NSLOT):
        wait_out(sl)


def _gather_pair(xs, xr, sidx3, ridx3):
    kfn = pl.kernel(
        _gather_body,
        out_type=(jax.ShapeDtypeStruct((E, D), jnp.int32),
                  jax.ShapeDtypeStruct((E, D), jnp.int32)),
        mesh=plsc.VectorSubcoreMesh(core_axis_name="c", subcore_axis_name="s"),
        scratch_types=(
            [pltpu.VMEM((NCHUNKG, CHUNK), jnp.int32)] * 2
            + [pltpu.VMEM((CHUNK, D), jnp.int32)] * (2 * NSLOT)
            + [pltpu.SemaphoreType.DMA] * (4 * NSLOT)
        ),
    )
    return kfn(xs, xr, sidx3, ridx3)


def _edge_body(ea_ref, gs_ref, gr_ref, we_ref, be_ref, ge_ref, bne_ref,
               w2_ref, b2_ref, out_ref):
    gs = gs_ref[...]
    gr = gr_ref[...]
    himask = jnp.int32(-65536)
    lo = (jax.lax.bitcast_convert_type(jnp.left_shift(gs, 16), jnp.float32)
          + jax.lax.bitcast_convert_type(jnp.left_shift(gr, 16), jnp.float32))
    hi = (jax.lax.bitcast_convert_type(jnp.bitwise_and(gs, himask),
                                       jnp.float32)
          + jax.lax.bitcast_convert_type(jnp.bitwise_and(gr, himask),
                                         jnp.float32))
    hpre = (jnp.concatenate([lo, hi], axis=-1)
            + jnp.dot(ea_ref[...], we_ref[...],
                      preferred_element_type=jnp.float32)
            + be_ref[...])
    mu = jnp.mean(hpre, axis=-1, keepdims=True)
    var = jnp.mean((hpre - mu) ** 2, axis=-1, keepdims=True)
    hn = (hpre - mu) * lax.rsqrt(var + 1e-5) * ge_ref[...] + bne_ref[...]
    h = jnp.maximum(hn, 0.0)
    out_ref[...] = (jnp.dot(h, w2_ref[...], preferred_element_type=jnp.float32)
                    + b2_ref[...])


def _edge_mlp(part, edge_attr, gs, gr, we, be, ge, bne, w2, b2):
    blk = 2000
    off = part * (EC // blk)
    return pl.pallas_call(
        _edge_body,
        grid=(EC // blk,),
        in_specs=[
            pl.BlockSpec((blk, D), lambda i: (i + off, 0)),
            pl.BlockSpec((blk, D), lambda i: (i + off, 0)),
            pl.BlockSpec((blk, D), lambda i: (i + off, 0)),
            pl.BlockSpec((D, H), lambda i: (0, 0)),
            pl.BlockSpec((1, H), lambda i: (0, 0)),
            pl.BlockSpec((1, H), lambda i: (0, 0)),
            pl.BlockSpec((1, H), lambda i: (0, 0)),
            pl.BlockSpec((H, D), lambda i: (0, 0)),
            pl.BlockSpec((1, D), lambda i: (0, 0)),
        ],
        out_specs=pl.BlockSpec((blk, D), lambda i: (i, 0)),
        out_shape=jax.ShapeDtypeStruct((EC, D), jnp.float32),
    )(edge_attr, gs, gr, we, be, ge, bne, w2, b2)


def _scatter_body(upd_hbm, ridx_hbm, zeros_hbm, out_hbm,
                  idx_all, upd0, upd1, acc_sh, sem_u0, sem_u1):
    c = lax.axis_index("c")
    s = lax.axis_index("s")
    wid = c * NS + s
    tile_base = wid * EPT
    pltpu.sync_copy(ridx_hbm.at[wid], idx_all)
    slab = pl.multiple_of(s * SLAB, 8)
    pltpu.sync_copy(zeros_hbm.at[pl.ds(slab, SLAB)],
                    acc_sh.at[pl.ds(slab, SLAB)])

    @pl.when(s == 0)
    def _init_tail():
        pltpu.sync_copy(zeros_hbm.at[pl.ds(SLAB * NS, TAIL)],
                        acc_sh.at[pl.ds(SLAB * NS, TAIL)])

    plsc.subcore_barrier()

    slots = ((upd0, sem_u0), (upd1, sem_u1))

    def issue(b, sl):
        buf, sem = slots[sl]
        base = pl.multiple_of(tile_base + b * SBLK, CHUNK)
        pltpu.async_copy(upd_hbm.at[pl.ds(base, SBLK)], buf, sem)

    def process(b, sl):
        buf, sem = slots[sl]
        pltpu.make_async_copy(upd_hbm.at[pl.ds(0, SBLK)], buf, sem).wait()
        for k in range(SCPB):
            pltpu.sync_copy(buf.at[pl.ds(k * CHUNK, CHUNK)],
                            acc_sh.at[idx_all.at[b * SCPB + k]], add=True)

        @pl.when(b + 2 < NSBLK)
        def _issue_next():
            issue(b + 2, sl)

    issue(0, 0)
    issue(1, 1)

    def pair(t, carry):
        process(2 * t, 0)
        process(2 * t + 1, 1)
        return carry

    lax.fori_loop(0, NSBLK // 2, pair, 0, unroll=False)
    process(NSBLK - 1, 0)

    plsc.subcore_barrier()
    pltpu.sync_copy(acc_sh.at[pl.ds(slab, SLAB)],
                    out_hbm.at[c].at[pl.ds(slab, SLAB)])

    @pl.when(s == 0)
    def _out_tail():
        pltpu.sync_copy(acc_sh.at[pl.ds(SLAB * NS, TAIL)],
                        out_hbm.at[c].at[pl.ds(SLAB * NS, TAIL)])


def _scatter_partials(upd, ridx3, zeros):
    kfn = pl.kernel(
        _scatter_body,
        out_type=jax.ShapeDtypeStruct((NC, N, D), jnp.float32),
        mesh=plsc.VectorSubcoreMesh(core_axis_name="c", subcore_axis_name="s"),
        scratch_types=[
            pltpu.VMEM((NCHUNK, CHUNK), jnp.int32),
            pltpu.VMEM((SBLK, D), jnp.float32),
            pltpu.VMEM((SBLK, D), jnp.float32),
            pltpu.VMEM_SHARED((N, D), jnp.float32),
            pltpu.SemaphoreType.DMA,
            pltpu.SemaphoreType.DMA,
        ],
    )
    return kfn(upd, ridx3, zeros)


def _node_body(x_ref, a0_ref, a1_ref, a2_ref, a3_ref, wx_ref, wa_ref,
               bn_ref, gn_ref, bnn_ref, w2_ref, b2_ref, out_ref):
    xb = x_ref[...]
    agg = ((a0_ref[...] + a1_ref[...]) + (a2_ref[...] + a3_ref[...]))
    hpre = (jnp.dot(xb, wx_ref[...], preferred_element_type=jnp.float32)
            + jnp.dot(agg, wa_ref[...], preferred_element_type=jnp.float32)
            + bn_ref[...])
    mu = jnp.mean(hpre, axis=-1, keepdims=True)
    var = jnp.mean((hpre - mu) ** 2, axis=-1, keepdims=True)
    hn = (hpre - mu) * lax.rsqrt(var + 1e-5) * gn_ref[...] + bnn_ref[...]
    h = jnp.maximum(hn, 0.0)
    out_ref[...] = (xb
                    + jnp.dot(h, w2_ref[...],
                              preferred_element_type=jnp.float32)
                    + b2_ref[...])


def _node_mlp(x, aggs, wx, wa, bn, gn, bnn, w2, b2):
    blk = 1000
    return pl.pallas_call(
        _node_body,
        grid=(N // blk,),
        in_specs=(
            [pl.BlockSpec((blk, D), lambda i: (i, 0))] * 5
            + [
                pl.BlockSpec((D, H), lambda i: (0, 0)),
                pl.BlockSpec((D, H), lambda i: (0, 0)),
                pl.BlockSpec((1, H), lambda i: (0, 0)),
                pl.BlockSpec((1, H), lambda i: (0, 0)),
                pl.BlockSpec((1, H), lambda i: (0, 0)),
                pl.BlockSpec((H, D), lambda i: (0, 0)),
                pl.BlockSpec((1, D), lambda i: (0, 0)),
            ]
        ),
        out_specs=pl.BlockSpec((blk, D), lambda i: (i, 0)),
        out_shape=jax.ShapeDtypeStruct((N, D), jnp.float32),
    )(x, *aggs, wx, wa, bn, gn, bnn, w2, b2)


def kernel(x, edge_index, edge_attr, We1, be1, ge1, bne1, We2, be2,
           Wn1, bn1, gn1, bnn1, Wn2, bn2):
    sidxg = edge_index[0].astype(jnp.int32).reshape(NW, NCHUNKG, CHUNK)
    ridxg = edge_index[1].astype(jnp.int32).reshape(NW, NCHUNKG, CHUNK)
    ridx = edge_index[1].astype(jnp.int32).reshape(P, NW, NCHUNK, CHUNK)

    we_e = We1[:D]
    we_s = We1[D:2 * D]
    we_r = We1[2 * D:]
    wn_x = Wn1[:D]
    wn_a = Wn1[D:]

    xs, xr = _pre_project(x, we_s, we_r)
    zeros = jnp.zeros((N, D), jnp.float32)

    gs, gr = _gather_pair(xs, xr, sidxg, ridxg)
    aggs = []
    for p in range(P):
        upd = _edge_mlp(p, edge_attr, gs, gr,
                        we_e, be1.reshape(1, H), ge1.reshape(1, H),
                        bne1.reshape(1, H), We2, be2.reshape(1, D))
        partials = _scatter_partials(upd, ridx[p], zeros)
        aggs.append(partials[0])
        aggs.append(partials[1])

    out = _node_mlp(x, aggs,
                    wn_x, wn_a, bn1.reshape(1, H), gn1.reshape(1, H),
                    bnn1.reshape(1, H), Wn2, bn2.reshape(1, D))
    return out


# speedup vs baseline: 1.1184x; 1.1184x over previous
import functools

import jax
import jax.numpy as jnp
from jax import lax
from jax.experimental import pallas as pl
from jax.experimental.pallas import tpu as pltpu
from jax.experimental.pallas import tpu_sc as plsc

N = 10000
E = 320000
D = 128
H = 256

NC = 2
NS = 16
NW = NC * NS

P = 2
EC = E // P
EPT = EC // NW
CHUNK = 40
NCHUNK = EPT // CHUNK
SBLK = 40
SCPB = SBLK // CHUNK
NSBLK = EPT // SBLK
SLAB = 624
TAIL = N - SLAB * NS


def _pack_pair(y):
    yb = y.astype(jnp.bfloat16)
    lo = jax.lax.bitcast_convert_type(yb[:, :D], jnp.uint16).astype(jnp.uint32)
    hi = jax.lax.bitcast_convert_type(yb[:, D:], jnp.uint16).astype(jnp.uint32)
    return jax.lax.bitcast_convert_type(lo | (hi << 16), jnp.int32)


def _pre_body(x_ref, ws_ref, wr_ref, xs_ref, xr_ref):
    xb = x_ref[...]
    xs_ref[...] = _pack_pair(
        jnp.dot(xb, ws_ref[...], preferred_element_type=jnp.float32))
    xr_ref[...] = _pack_pair(
        jnp.dot(xb, wr_ref[...], preferred_element_type=jnp.float32))


def _pre_project(x, ws, wr):
    blk = 1000
    return pl.pallas_call(
        _pre_body,
        grid=(N // blk,),
        in_specs=[
            pl.BlockSpec((blk, D), lambda i: (i, 0)),
            pl.BlockSpec((D, H), lambda i: (0, 0)),
            pl.BlockSpec((D, H), lambda i: (0, 0)),
        ],
        out_specs=[
            pl.BlockSpec((blk, D), lambda i: (i, 0)),
            pl.BlockSpec((blk, D), lambda i: (i, 0)),
        ],
        out_shape=[
            jax.ShapeDtypeStruct((N, D), jnp.int32),
            jax.ShapeDtypeStruct((N, D), jnp.int32),
        ],
    )(x, ws, wr)


NSLOT = 4


def _gather_body(xs_hbm, xr_hbm, sidx_hbm, ridx_hbm, outs_hbm, outr_hbm,
                 sidx_all, ridx_all,
                 ba0, bb0, ba1, bb1, ba2, bb2, ba3, bb3,
                 sa0, sb0, soa0, sob0, sa1, sb1, soa1, sob1,
                 sa2, sb2, soa2, sob2, sa3, sb3, soa3, sob3):
    c = lax.axis_index("c")
    s = lax.axis_index("s")
    wid = s * NC + c
    tile_base = wid * EPT
    pltpu.sync_copy(sidx_hbm.at[wid], sidx_all)
    pltpu.sync_copy(ridx_hbm.at[wid], ridx_all)

    slots = ((ba0, bb0, sa0, sb0, soa0, sob0),
             (ba1, bb1, sa1, sb1, soa1, sob1),
             (ba2, bb2, sa2, sb2, soa2, sob2),
             (ba3, bb3, sa3, sb3, soa3, sob3))

    def issue_gather(q, sl):
        ba, bb, sa, sb, _, _ = slots[sl]
        pltpu.async_copy(xs_hbm.at[sidx_all.at[q]], ba, sa)
        pltpu.async_copy(xr_hbm.at[ridx_all.at[q]], bb, sb)

    def wait_gather(sl):
        ba, bb, sa, sb, _, _ = slots[sl]
        pltpu.make_async_copy(xs_hbm.at[sidx_all.at[0]], ba, sa).wait()
        pltpu.make_async_copy(xr_hbm.at[ridx_all.at[0]], bb, sb).wait()

    def issue_out(q, sl):
        ba, bb, _, _, soa, sob = slots[sl]
        base = pl.multiple_of(tile_base + q * CHUNK, CHUNK)
        pltpu.async_copy(ba, outs_hbm.at[pl.ds(base, CHUNK)], soa)
        pltpu.async_copy(bb, outr_hbm.at[pl.ds(base, CHUNK)], sob)

    def wait_out(sl):
        ba, bb, _, _, soa, sob = slots[sl]
        pltpu.make_async_copy(ba, outs_hbm.at[pl.ds(0, CHUNK)], soa).wait()
        pltpu.make_async_copy(bb, outr_hbm.at[pl.ds(0, CHUNK)], sob).wait()

    for q0 in range(NSLOT - 1):
        issue_gather(q0, q0)

    def group(t, carry):
        for s4 in range(NSLOT):
            q = NSLOT * t + s4
            prev = (s4 - 1) % NSLOT

            @pl.when(q < NCHUNK)
            def _visit():
                wait_gather(s4)
                issue_out(q, s4)

                @pl.when((q >= 1) & (q + NSLOT - 1 < NCHUNK))
                def _reissue():
                    wait_out(prev)
                    issue_gather(q + NSLOT - 1, prev)

                @pl.when(q == 0)
                def _first():
                    issue_gather(NSLOT - 1, prev)

        return carry

    lax.fori_loop(0, (NCHUNK + NSLOT - 1) // NSLOT, group, 0, unroll=False)
    for sl in range(NSLOT):
        wait_out(sl)


def _gather_pair(xs, xr, sidx3, ridx3):
    kfn = pl.kernel(
        _gather_body,
        out_type=(jax.ShapeDtypeStruct((EC, D), jnp.int32),
                  jax.ShapeDtypeStruct((EC, D), jnp.int32)),
        mesh=plsc.VectorSubcoreMesh(core_axis_name="c", subcore_axis_name="s"),
        scratch_types=(
            [pltpu.VMEM((NCHUNK, CHUNK), jnp.int32)] * 2
            + [pltpu.VMEM((CHUNK, D), jnp.int32)] * (2 * NSLOT)
            + [pltpu.SemaphoreType.DMA] * (4 * NSLOT)
        ),
    )
    return kfn(xs, xr, sidx3, ridx3)


def _edge_body(ea_ref, gs_ref, gr_ref, we_ref, be_ref, ge_ref, bne_ref,
               w2_ref, b2_ref, out_ref):
    gs = gs_ref[...]
    gr = gr_ref[...]
    himask = jnp.int32(-65536)
    lo = (jax.lax.bitcast_convert_type(jnp.left_shift(gs, 16), jnp.float32)
          + jax.lax.bitcast_convert_type(jnp.left_shift(gr, 16), jnp.float32))
    hi = (jax.lax.bitcast_convert_type(jnp.bitwise_and(gs, himask),
                                       jnp.float32)
          + jax.lax.bitcast_convert_type(jnp.bitwise_and(gr, himask),
                                         jnp.float32))
    hpre = (jnp.concatenate([lo, hi], axis=-1)
            + jnp.dot(ea_ref[...], we_ref[...],
                      preferred_element_type=jnp.float32)
            + be_ref[...])
    mu = jnp.mean(hpre, axis=-1, keepdims=True)
    var = jnp.mean((hpre - mu) ** 2, axis=-1, keepdims=True)
    hn = (hpre - mu) * lax.rsqrt(var + 1e-5) * ge_ref[...] + bne_ref[...]
    h = jnp.maximum(hn, 0.0)
    out_ref[...] = (jnp.dot(h, w2_ref[...], preferred_element_type=jnp.float32)
                    + b2_ref[...])


def _edge_mlp(part, edge_attr, gs, gr, we, be, ge, bne, w2, b2):
    blk = 4000
    off = part * (EC // blk)
    return pl.pallas_call(
        _edge_body,
        grid=(EC // blk,),
        in_specs=[
            pl.BlockSpec((blk, D), lambda i: (i + off, 0)),
            pl.BlockSpec((blk, D), lambda i: (i, 0)),
            pl.BlockSpec((blk, D), lambda i: (i, 0)),
            pl.BlockSpec((D, H), lambda i: (0, 0)),
            pl.BlockSpec((1, H), lambda i: (0, 0)),
            pl.BlockSpec((1, H), lambda i: (0, 0)),
            pl.BlockSpec((1, H), lambda i: (0, 0)),
            pl.BlockSpec((H, D), lambda i: (0, 0)),
            pl.BlockSpec((1, D), lambda i: (0, 0)),
        ],
        out_specs=pl.BlockSpec((blk, D), lambda i: (i, 0)),
        out_shape=jax.ShapeDtypeStruct((EC, D), jnp.float32),
    )(edge_attr, gs, gr, we, be, ge, bne, w2, b2)


def _scatter_body(upd_hbm, ridx_hbm, zeros_hbm, outa_hbm, outb_hbm,
                  idx_all, upd0, upd1, acc_sh, sem_u0, sem_u1):
    c = lax.axis_index("c")
    s = lax.axis_index("s")
    wid = c * NS + s
    tile_base = wid * EPT
    pltpu.sync_copy(ridx_hbm.at[wid], idx_all)
    slab = pl.multiple_of(s * SLAB, 8)
    pltpu.sync_copy(zeros_hbm.at[pl.ds(slab, SLAB)],
                    acc_sh.at[pl.ds(slab, SLAB)])

    @pl.when(s == 0)
    def _init_tail():
        pltpu.sync_copy(zeros_hbm.at[pl.ds(SLAB * NS, TAIL)],
                        acc_sh.at[pl.ds(SLAB * NS, TAIL)])

    plsc.subcore_barrier()

    slots = ((upd0, sem_u0), (upd1, sem_u1))

    def issue(b, sl):
        buf, sem = slots[sl]
        base = pl.multiple_of(tile_base + b * SBLK, CHUNK)
        pltpu.async_copy(upd_hbm.at[pl.ds(base, SBLK)], buf, sem)

    def process(b, sl):
        buf, sem = slots[sl]
        pltpu.make_async_copy(upd_hbm.at[pl.ds(0, SBLK)], buf, sem).wait()
        for k in range(SCPB):
            pltpu.sync_copy(buf.at[pl.ds(k * CHUNK, CHUNK)],
                            acc_sh.at[idx_all.at[b * SCPB + k]], add=True)

        @pl.when(b + 2 < NSBLK)
        def _issue_next():
            issue(b + 2, sl)

    issue(0, 0)
    issue(1, 1)

    def pair(t, carry):
        process(2 * t, 0)
        process(2 * t + 1, 1)
        return carry

    lax.fori_loop(0, NSBLK // 2, pair, 0, unroll=False)
    process(NSBLK - 1, 0)

    plsc.subcore_barrier()

    @pl.when(c == 0)
    def _out_a():
        pltpu.sync_copy(acc_sh.at[pl.ds(slab, SLAB)],
                        outa_hbm.at[pl.ds(slab, SLAB)])

        @pl.when(s == 0)
        def _out_a_tail():
            pltpu.sync_copy(acc_sh.at[pl.ds(SLAB * NS, TAIL)],
                            outa_hbm.at[pl.ds(SLAB * NS, TAIL)])

    @pl.when(c == 1)
    def _out_b():
        pltpu.sync_copy(acc_sh.at[pl.ds(slab, SLAB)],
                        outb_hbm.at[pl.ds(slab, SLAB)])

        @pl.when(s == 0)
        def _out_b_tail():
            pltpu.sync_copy(acc_sh.at[pl.ds(SLAB * NS, TAIL)],
                            outb_hbm.at[pl.ds(SLAB * NS, TAIL)])


def _scatter_partials(upd, ridx3, zeros):
    kfn = pl.kernel(
        _scatter_body,
        out_type=(jax.ShapeDtypeStruct((N, D), jnp.float32),
                  jax.ShapeDtypeStruct((N, D), jnp.float32)),
        mesh=plsc.VectorSubcoreMesh(core_axis_name="c", subcore_axis_name="s"),
        scratch_types=[
            pltpu.VMEM((NCHUNK, CHUNK), jnp.int32),
            pltpu.VMEM((SBLK, D), jnp.float32),
            pltpu.VMEM((SBLK, D), jnp.float32),
            pltpu.VMEM_SHARED((N, D), jnp.float32),
            pltpu.SemaphoreType.DMA,
            pltpu.SemaphoreType.DMA,
        ],
    )
    return kfn(upd, ridx3, zeros)


def _node_body(x_ref, a0_ref, a1_ref, a2_ref, a3_ref, wx_ref, wa_ref,
               bn_ref, gn_ref, bnn_ref, w2_ref, b2_ref, out_ref):
    xb = x_ref[...]
    agg = ((a0_ref[...] + a1_ref[...]) + (a2_ref[...] + a3_ref[...]))
    hpre = (jnp.dot(xb, wx_ref[...], preferred_element_type=jnp.float32)
            + jnp.dot(agg, wa_ref[...], preferred_element_type=jnp.float32)
            + bn_ref[...])
    mu = jnp.mean(hpre, axis=-1, keepdims=True)
    var = jnp.mean((hpre - mu) ** 2, axis=-1, keepdims=True)
    hn = (hpre - mu) * lax.rsqrt(var + 1e-5) * gn_ref[...] + bnn_ref[...]
    h = jnp.maximum(hn, 0.0)
    out_ref[...] = (xb
                    + jnp.dot(h, w2_ref[...],
                              preferred_element_type=jnp.float32)
                    + b2_ref[...])


def _node_mlp(x, aggs, wx, wa, bn, gn, bnn, w2, b2):
    blk = 1000
    return pl.pallas_call(
        _node_body,
        grid=(N // blk,),
        in_specs=(
            [pl.BlockSpec((blk, D), lambda i: (i, 0))] * 5
            + [
                pl.BlockSpec((D, H), lambda i: (0, 0)),
                pl.BlockSpec((D, H), lambda i: (0, 0)),
                pl.BlockSpec((1, H), lambda i: (0, 0)),
                pl.BlockSpec((1, H), lambda i: (0, 0)),
                pl.BlockSpec((1, H), lambda i: (0, 0)),
                pl.BlockSpec((H, D), lambda i: (0, 0)),
                pl.BlockSpec((1, D), lambda i: (0, 0)),
            ]
        ),
        out_specs=pl.BlockSpec((blk, D), lambda i: (i, 0)),
        out_shape=jax.ShapeDtypeStruct((N, D), jnp.float32),
    )(x, *aggs, wx, wa, bn, gn, bnn, w2, b2)


def kernel(x, edge_index, edge_attr, We1, be1, ge1, bne1, We2, be2,
           Wn1, bn1, gn1, bnn1, Wn2, bn2):
    sidx = edge_index[0].astype(jnp.int32).reshape(P, NW, NCHUNK, CHUNK)
    ridx = edge_index[1].astype(jnp.int32).reshape(P, NW, NCHUNK, CHUNK)

    we_e = We1[:D]
    we_s = We1[D:2 * D]
    we_r = We1[2 * D:]
    wn_x = Wn1[:D]
    wn_a = Wn1[D:]

    xs, xr = _pre_project(x, we_s, we_r)
    zeros = jnp.zeros((N, D), jnp.float32)

    gpairs = [_gather_pair(xs, xr, sidx[p], ridx[p]) for p in range(P)]
    aggs = []
    for p in range(P):
        gs, gr = gpairs[p]
        upd = _edge_mlp(p, edge_attr, gs, gr,
                        we_e, be1.reshape(1, H), ge1.reshape(1, H),
                        bne1.reshape(1, H), We2, be2.reshape(1, D))
        pa, pb = _scatter_partials(upd, ridx[p], zeros)
        aggs.append(pa)
        aggs.append(pb)

    out = _node_mlp(x, aggs,
                    wn_x, wn_a, bn1.reshape(1, H), gn1.reshape(1, H),
                    bnn1.reshape(1, H), Wn2, bn2.reshape(1, D))
    return out


# speedup vs baseline: 1.1268x; 1.0076x over previous
import functools

import jax
import jax.numpy as jnp
from jax import lax
from jax.experimental import pallas as pl
from jax.experimental.pallas import tpu as pltpu
from jax.experimental.pallas import tpu_sc as plsc

N = 10000
E = 320000
D = 128
H = 256

NC = 2
NS = 16
NW = NC * NS

P = 2
EC = E // P
EPT = EC // NW
CHUNK = 40
NCHUNK = EPT // CHUNK
SBLK = 40
SCPB = SBLK // CHUNK
NSBLK = EPT // SBLK
SLAB = 624
TAIL = N - SLAB * NS


def _pack_pair(y):
    yb = y.astype(jnp.bfloat16)
    lo = jax.lax.bitcast_convert_type(yb[:, :D], jnp.uint16).astype(jnp.uint32)
    hi = jax.lax.bitcast_convert_type(yb[:, D:], jnp.uint16).astype(jnp.uint32)
    return jax.lax.bitcast_convert_type(lo | (hi << 16), jnp.int32)


def _pre_body(x_ref, ws_ref, wr_ref, xs_ref, xr_ref):
    xb = x_ref[...]
    xs_ref[...] = _pack_pair(
        jnp.dot(xb, ws_ref[...], preferred_element_type=jnp.float32))
    xr_ref[...] = _pack_pair(
        jnp.dot(xb, wr_ref[...], preferred_element_type=jnp.float32))


def _pre_project(x, ws, wr):
    blk = 2000
    return pl.pallas_call(
        _pre_body,
        grid=(N // blk,),
        in_specs=[
            pl.BlockSpec((blk, D), lambda i: (i, 0)),
            pl.BlockSpec((D, H), lambda i: (0, 0)),
            pl.BlockSpec((D, H), lambda i: (0, 0)),
        ],
        out_specs=[
            pl.BlockSpec((blk, D), lambda i: (i, 0)),
            pl.BlockSpec((blk, D), lambda i: (i, 0)),
        ],
        out_shape=[
            jax.ShapeDtypeStruct((N, D), jnp.int32),
            jax.ShapeDtypeStruct((N, D), jnp.int32),
        ],
    )(x, ws, wr)


NSLOT = 4


def _gather_body(xs_hbm, xr_hbm, sidx_hbm, ridx_hbm, outs_hbm, outr_hbm,
                 sidx_all, ridx_all,
                 ba0, bb0, ba1, bb1, ba2, bb2, ba3, bb3,
                 sa0, sb0, soa0, sob0, sa1, sb1, soa1, sob1,
                 sa2, sb2, soa2, sob2, sa3, sb3, soa3, sob3):
    c = lax.axis_index("c")
    s = lax.axis_index("s")
    wid = s * NC + c
    tile_base = wid * EPT
    pltpu.sync_copy(sidx_hbm.at[wid], sidx_all)
    pltpu.sync_copy(ridx_hbm.at[wid], ridx_all)

    slots = ((ba0, bb0, sa0, sb0, soa0, sob0),
             (ba1, bb1, sa1, sb1, soa1, sob1),
             (ba2, bb2, sa2, sb2, soa2, sob2),
             (ba3, bb3, sa3, sb3, soa3, sob3))

    def issue_gather(q, sl):
        ba, bb, sa, sb, _, _ = slots[sl]
        pltpu.async_copy(xs_hbm.at[sidx_all.at[q]], ba, sa)
        pltpu.async_copy(xr_hbm.at[ridx_all.at[q]], bb, sb)

    def wait_gather(sl):
        ba, bb, sa, sb, _, _ = slots[sl]
        pltpu.make_async_copy(xs_hbm.at[sidx_all.at[0]], ba, sa).wait()
        pltpu.make_async_copy(xr_hbm.at[ridx_all.at[0]], bb, sb).wait()

    def issue_out(q, sl):
        ba, bb, _, _, soa, sob = slots[sl]
        base = pl.multiple_of(tile_base + q * CHUNK, CHUNK)
        pltpu.async_copy(ba, outs_hbm.at[pl.ds(base, CHUNK)], soa)
        pltpu.async_copy(bb, outr_hbm.at[pl.ds(base, CHUNK)], sob)

    def wait_out(sl):
        ba, bb, _, _, soa, sob = slots[sl]
        pltpu.make_async_copy(ba, outs_hbm.at[pl.ds(0, CHUNK)], soa).wait()
        pltpu.make_async_copy(bb, outr_hbm.at[pl.ds(0, CHUNK)], sob).wait()

    for q0 in range(NSLOT - 1):
        issue_gather(q0, q0)

    def group(t, carry):
        for s4 in range(NSLOT):
            q = NSLOT * t + s4
            prev = (s4 - 1) % NSLOT

            @pl.when(q < NCHUNK)
            def _visit():
                wait_gather(s4)
                issue_out(q, s4)

                @pl.when((q >= 1) & (q + NSLOT - 1 < NCHUNK))
                def _reissue():
                    wait_out(prev)
                    issue_gather(q + NSLOT - 1, prev)

                @pl.when(q == 0)
                def _first():
                    issue_gather(NSLOT - 1, prev)

        return carry

    lax.fori_loop(0, (NCHUNK + NSLOT - 1) // NSLOT, group, 0, unroll=False)
    for sl in range(NSLOT):
        wait_out(sl)


def _gather_pair(xs, xr, sidx3, ridx3):
    kfn = pl.kernel(
        _gather_body,
        out_type=(jax.ShapeDtypeStruct((EC, D), jnp.int32),
                  jax.ShapeDtypeStruct((EC, D), jnp.int32)),
        mesh=plsc.VectorSubcoreMesh(core_axis_name="c", subcore_axis_name="s"),
        scratch_types=(
            [pltpu.VMEM((NCHUNK, CHUNK), jnp.int32)] * 2
            + [pltpu.VMEM((CHUNK, D), jnp.int32)] * (2 * NSLOT)
            + [pltpu.SemaphoreType.DMA] * (4 * NSLOT)
        ),
    )
    return kfn(xs, xr, sidx3, ridx3)


def _edge_body(ea_ref, gs_ref, gr_ref, we_ref, be_ref, ge_ref, bne_ref,
               w2_ref, b2_ref, out_ref):
    gs = gs_ref[...]
    gr = gr_ref[...]
    himask = jnp.int32(-65536)
    lo = (jax.lax.bitcast_convert_type(jnp.left_shift(gs, 16), jnp.float32)
          + jax.lax.bitcast_convert_type(jnp.left_shift(gr, 16), jnp.float32))
    hi = (jax.lax.bitcast_convert_type(jnp.bitwise_and(gs, himask),
                                       jnp.float32)
          + jax.lax.bitcast_convert_type(jnp.bitwise_and(gr, himask),
                                         jnp.float32))
    hpre = (jnp.concatenate([lo, hi], axis=-1)
            + jnp.dot(ea_ref[...], we_ref[...],
                      preferred_element_type=jnp.float32)
            + be_ref[...])
    mu = jnp.mean(hpre, axis=-1, keepdims=True)
    var = jnp.mean((hpre - mu) ** 2, axis=-1, keepdims=True)
    hn = (hpre - mu) * lax.rsqrt(var + 1e-5) * ge_ref[...] + bne_ref[...]
    h = jnp.maximum(hn, 0.0)
    out_ref[...] = (jnp.dot(h, w2_ref[...], preferred_element_type=jnp.float32)
                    + b2_ref[...])


def _edge_mlp(part, edge_attr, gs, gr, we, be, ge, bne, w2, b2):
    blk = 8000
    off = part * (EC // blk)
    return pl.pallas_call(
        _edge_body,
        grid=(EC // blk,),
        in_specs=[
            pl.BlockSpec((blk, D), lambda i: (i + off, 0)),
            pl.BlockSpec((blk, D), lambda i: (i, 0)),
            pl.BlockSpec((blk, D), lambda i: (i, 0)),
            pl.BlockSpec((D, H), lambda i: (0, 0)),
            pl.BlockSpec((1, H), lambda i: (0, 0)),
            pl.BlockSpec((1, H), lambda i: (0, 0)),
            pl.BlockSpec((1, H), lambda i: (0, 0)),
            pl.BlockSpec((H, D), lambda i: (0, 0)),
            pl.BlockSpec((1, D), lambda i: (0, 0)),
        ],
        out_specs=pl.BlockSpec((blk, D), lambda i: (i, 0)),
        out_shape=jax.ShapeDtypeStruct((EC, D), jnp.float32),
    )(edge_attr, gs, gr, we, be, ge, bne, w2, b2)


def _scatter_body(upd_hbm, ridx_hbm, zeros_hbm, outa_hbm, outb_hbm,
                  idx_all, upd0, upd1, acc_sh, sem_u0, sem_u1):
    c = lax.axis_index("c")
    s = lax.axis_index("s")
    wid = c * NS + s
    tile_base = wid * EPT
    pltpu.sync_copy(ridx_hbm.at[wid], idx_all)
    slab = pl.multiple_of(s * SLAB, 8)
    pltpu.sync_copy(zeros_hbm.at[pl.ds(slab, SLAB)],
                    acc_sh.at[pl.ds(slab, SLAB)])

    @pl.when(s == 0)
    def _init_tail():
        pltpu.sync_copy(zeros_hbm.at[pl.ds(SLAB * NS, TAIL)],
                        acc_sh.at[pl.ds(SLAB * NS, TAIL)])

    plsc.subcore_barrier()

    slots = ((upd0, sem_u0), (upd1, sem_u1))

    def issue(b, sl):
        buf, sem = slots[sl]
        base = pl.multiple_of(tile_base + b * SBLK, CHUNK)
        pltpu.async_copy(upd_hbm.at[pl.ds(base, SBLK)], buf, sem)

    def process(b, sl):
        buf, sem = slots[sl]
        pltpu.make_async_copy(upd_hbm.at[pl.ds(0, SBLK)], buf, sem).wait()
        for k in range(SCPB):
            pltpu.sync_copy(buf.at[pl.ds(k * CHUNK, CHUNK)],
                            acc_sh.at[idx_all.at[b * SCPB + k]], add=True)

        @pl.when(b + 2 < NSBLK)
        def _issue_next():
            issue(b + 2, sl)

    issue(0, 0)
    issue(1, 1)

    def pair(t, carry):
        process(2 * t, 0)
        process(2 * t + 1, 1)
        return carry

    lax.fori_loop(0, NSBLK // 2, pair, 0, unroll=False)
    process(NSBLK - 1, 0)

    plsc.subcore_barrier()

    @pl.when(c == 0)
    def _out_a():
        pltpu.sync_copy(acc_sh.at[pl.ds(slab, SLAB)],
                        outa_hbm.at[pl.ds(slab, SLAB)])

        @pl.when(s == 0)
        def _out_a_tail():
            pltpu.sync_copy(acc_sh.at[pl.ds(SLAB * NS, TAIL)],
                            outa_hbm.at[pl.ds(SLAB * NS, TAIL)])

    @pl.when(c == 1)
    def _out_b():
        pltpu.sync_copy(acc_sh.at[pl.ds(slab, SLAB)],
                        outb_hbm.at[pl.ds(slab, SLAB)])

        @pl.when(s == 0)
        def _out_b_tail():
            pltpu.sync_copy(acc_sh.at[pl.ds(SLAB * NS, TAIL)],
                            outb_hbm.at[pl.ds(SLAB * NS, TAIL)])


def _scatter_partials(upd, ridx3, zeros):
    kfn = pl.kernel(
        _scatter_body,
        out_type=(jax.ShapeDtypeStruct((N, D), jnp.float32),
                  jax.ShapeDtypeStruct((N, D), jnp.float32)),
        mesh=plsc.VectorSubcoreMesh(core_axis_name="c", subcore_axis_name="s"),
        scratch_types=[
            pltpu.VMEM((NCHUNK, CHUNK), jnp.int32),
            pltpu.VMEM((SBLK, D), jnp.float32),
            pltpu.VMEM((SBLK, D), jnp.float32),
            pltpu.VMEM_SHARED((N, D), jnp.float32),
            pltpu.SemaphoreType.DMA,
            pltpu.SemaphoreType.DMA,
        ],
    )
    return kfn(upd, ridx3, zeros)


def _node_body(x_ref, a0_ref, a1_ref, a2_ref, a3_ref, wx_ref, wa_ref,
               bn_ref, gn_ref, bnn_ref, w2_ref, b2_ref, out_ref):
    xb = x_ref[...]
    agg = ((a0_ref[...] + a1_ref[...]) + (a2_ref[...] + a3_ref[...]))
    hpre = (jnp.dot(xb, wx_ref[...], preferred_element_type=jnp.float32)
            + jnp.dot(agg, wa_ref[...], preferred_element_type=jnp.float32)
            + bn_ref[...])
    mu = jnp.mean(hpre, axis=-1, keepdims=True)
    var = jnp.mean((hpre - mu) ** 2, axis=-1, keepdims=True)
    hn = (hpre - mu) * lax.rsqrt(var + 1e-5) * gn_ref[...] + bnn_ref[...]
    h = jnp.maximum(hn, 0.0)
    out_ref[...] = (xb
                    + jnp.dot(h, w2_ref[...],
                              preferred_element_type=jnp.float32)
                    + b2_ref[...])


def _node_mlp(x, aggs, wx, wa, bn, gn, bnn, w2, b2):
    blk = 2000
    return pl.pallas_call(
        _node_body,
        grid=(N // blk,),
        in_specs=(
            [pl.BlockSpec((blk, D), lambda i: (i, 0))] * 5
            + [
                pl.BlockSpec((D, H), lambda i: (0, 0)),
                pl.BlockSpec((D, H), lambda i: (0, 0)),
                pl.BlockSpec((1, H), lambda i: (0, 0)),
                pl.BlockSpec((1, H), lambda i: (0, 0)),
                pl.BlockSpec((1, H), lambda i: (0, 0)),
                pl.BlockSpec((H, D), lambda i: (0, 0)),
                pl.BlockSpec((1, D), lambda i: (0, 0)),
            ]
        ),
        out_specs=pl.BlockSpec((blk, D), lambda i: (i, 0)),
        out_shape=jax.ShapeDtypeStruct((N, D), jnp.float32),
    )(x, *aggs, wx, wa, bn, gn, bnn, w2, b2)


def kernel(x, edge_index, edge_attr, We1, be1, ge1, bne1, We2, be2,
           Wn1, bn1, gn1, bnn1, Wn2, bn2):
    sidx = edge_index[0].astype(jnp.int32).reshape(P, NW, NCHUNK, CHUNK)
    ridx = edge_index[1].astype(jnp.int32).reshape(P, NW, NCHUNK, CHUNK)

    we_e = We1[:D]
    we_s = We1[D:2 * D]
    we_r = We1[2 * D:]
    wn_x = Wn1[:D]
    wn_a = Wn1[D:]

    xs, xr = _pre_project(x, we_s, we_r)
    zeros = jnp.zeros((N, D), jnp.float32)

    gpairs = [_gather_pair(xs, xr, sidx[p], ridx[p]) for p in range(P)]
    aggs = []
    for p in range(P):
        gs, gr = gpairs[p]
        upd = _edge_mlp(p, edge_attr, gs, gr,
                        we_e, be1.reshape(1, H), ge1.reshape(1, H),
                        bne1.reshape(1, H), We2, be2.reshape(1, D))
        pa, pb = _scatter_partials(upd, ridx[p], zeros)
        aggs.append(pa)
        aggs.append(pb)

    out = _node_mlp(x, aggs,
                    wn_x, wn_a, bn1.reshape(1, H), gn1.reshape(1, H),
                    bnn1.reshape(1, H), Wn2, bn2.reshape(1, D))
    return out
